# Initial kernel scaffold; baseline (speedup 1.0000x reference)
import functools

import jax
import jax.numpy as jnp
from jax import lax
from jax.experimental import pallas as pl
from jax.experimental.pallas import tpu as pltpu
from jax.experimental.pallas import tpu_sc as plsc

BATCH = 4096
SEQ = 200
EMBED = 32
TOTAL = BATCH * SEQ

NUM_CORES = 2
NUM_SUBCORES = 16
NW = NUM_CORES * NUM_SUBCORES
PER_WORKER = TOTAL // NW
CHUNK = 1024
NCHUNK = PER_WORKER // CHUNK


def _emb_body(idx_hbm, table_hbm, out_hbm, idx_v, rows_v, sem):
    wid = lax.axis_index("s") * NUM_CORES + lax.axis_index("c")
    base = wid * PER_WORKER

    def body(i, carry):
        off = base + i * CHUNK
        pltpu.sync_copy(idx_hbm.at[pl.ds(off, CHUNK)], idx_v)
        pltpu.async_copy(table_hbm.at[idx_v], rows_v, sem).wait()
        pltpu.sync_copy(rows_v, out_hbm.at[pl.ds(off, CHUNK)])
        return carry

    lax.fori_loop(0, NCHUNK, body, 0)


@jax.jit
def _embedding_lookup(x_flat, table):
    mesh = plsc.VectorSubcoreMesh(core_axis_name="c", subcore_axis_name="s")
    kern = functools.partial(
        pl.kernel,
        mesh=mesh,
        out_type=jax.ShapeDtypeStruct((TOTAL, EMBED), jnp.float32),
        scratch_types=[
            pltpu.VMEM((CHUNK,), jnp.int32),
            pltpu.VMEM((CHUNK, EMBED), jnp.float32),
            pltpu.SemaphoreType.DMA,
        ],
    )(_emb_body)
    return kern(x_flat, table)


def kernel(x, W_embed):
    x_flat = x.reshape(TOTAL).astype(jnp.int32)
    out = _embedding_lookup(x_flat, W_embed)
    return out.reshape(BATCH, SEQ, EMBED)


# speedup vs baseline: 1.4598x; 1.4598x over previous
import functools

import jax
import jax.numpy as jnp
from jax import lax
from jax.experimental import pallas as pl
from jax.experimental.pallas import tpu as pltpu
from jax.experimental.pallas import tpu_sc as plsc

BATCH = 4096
SEQ = 200
EMBED = 32
TOTAL = BATCH * SEQ

NUM_CORES = 2
NUM_SUBCORES = 16
NW = NUM_CORES * NUM_SUBCORES
PER_WORKER = TOTAL // NW
CHUNK = 1024
NCHUNK = PER_WORKER // CHUNK


def _emb_body(idx_hbm, table_hbm, out_hbm, idx_v, rows_v, sem):
    wid = lax.axis_index("s") * NUM_CORES + lax.axis_index("c")
    base = wid * PER_WORKER

    def body(i, carry):
        off = base + i * CHUNK
        pltpu.sync_copy(idx_hbm.at[pl.ds(off, CHUNK)], idx_v)
        pltpu.async_copy(table_hbm.at[idx_v], rows_v, sem).wait()
        pltpu.sync_copy(rows_v, out_hbm.at[pl.ds(off, CHUNK)])
        return carry

    lax.fori_loop(0, NCHUNK, body, 0)


@jax.jit
def _embedding_lookup(x_flat, table):
    mesh = plsc.VectorSubcoreMesh(core_axis_name="c", subcore_axis_name="s")
    kern = functools.partial(
        pl.kernel,
        mesh=mesh,
        out_type=jax.ShapeDtypeStruct((TOTAL, EMBED), jnp.float32),
        scratch_types=[
            pltpu.VMEM((CHUNK,), jnp.int32),
            pltpu.VMEM((CHUNK, EMBED), jnp.float32),
            pltpu.SemaphoreType.DMA,
        ],
        compiler_params=pltpu.CompilerParams(use_tc_tiling_on_sc=False),
    )(_emb_body)
    return kern(x_flat, table)


def kernel(x, W_embed):
    x_flat = x.reshape(TOTAL).astype(jnp.int32)
    out = _embedding_lookup(x_flat, W_embed)
    return out.reshape(BATCH, SEQ, EMBED)


# speedup vs baseline: 1.4982x; 1.0263x over previous
import functools

import jax
import jax.numpy as jnp
from jax import lax
from jax.experimental import pallas as pl
from jax.experimental.pallas import tpu as pltpu
from jax.experimental.pallas import tpu_sc as plsc

BATCH = 4096
SEQ = 200
EMBED = 32
TOTAL = BATCH * SEQ

NUM_CORES = 2
NUM_SUBCORES = 16
NW = NUM_CORES * NUM_SUBCORES
PER_WORKER = TOTAL // NW
NB = 2
CHUNK = 1280
NCHUNK = PER_WORKER // CHUNK


def _emb_body(idx_hbm, table_hbm, out_hbm, idx_v, *scr):
    rows = scr[:NB]
    gsem = scr[NB:2 * NB]
    ssem = scr[2 * NB:3 * NB]

    wid = lax.axis_index("s") * NUM_CORES + lax.axis_index("c")
    base = wid * PER_WORKER

    pltpu.sync_copy(idx_hbm.at[pl.ds(base, PER_WORKER)], idx_v)

    def gdesc(i, b):
        return pltpu.make_async_copy(
            table_hbm.at[idx_v.at[pl.ds(i * CHUNK, CHUNK)]], rows[b], gsem[b])

    def sdesc(i, b):
        return pltpu.make_async_copy(
            rows[b], out_hbm.at[pl.ds(base + i * CHUNK, CHUNK)], ssem[b])

    for k in range(NB - 1):
        gdesc(k, k).start()

    @pl.loop(0, NCHUNK, step=NB)
    def _(i):
        for b in range(NB):
            j = i + b
            gdesc(0, b).wait()
            sdesc(j, b).start()
            nxt = j + NB - 1
            pb = (b + NB - 1) % NB

            @pl.when(jnp.logical_and(nxt >= NB, nxt <= NCHUNK - 1))
            def _():
                sdesc(0, pb).wait()
                gdesc(nxt, pb).start()

            @pl.when(jnp.logical_and(nxt < NB, nxt <= NCHUNK - 1))
            def _():
                gdesc(nxt, pb).start()

    for b in range(NB):
        sdesc(0, b).wait()


@jax.jit
def _embedding_lookup(x_flat, table):
    mesh = plsc.VectorSubcoreMesh(core_axis_name="c", subcore_axis_name="s")
    kern = functools.partial(
        pl.kernel,
        mesh=mesh,
        out_type=jax.ShapeDtypeStruct((TOTAL, EMBED), jnp.float32),
        scratch_types=(
            [pltpu.VMEM((PER_WORKER,), jnp.int32)]
            + [pltpu.VMEM((CHUNK, EMBED), jnp.float32)] * NB
            + [pltpu.SemaphoreType.DMA] * (2 * NB)
        ),
        compiler_params=pltpu.CompilerParams(use_tc_tiling_on_sc=False),
    )(_emb_body)
    return kern(x_flat, table)


def kernel(x, W_embed):
    x_flat = x.reshape(TOTAL).astype(jnp.int32)
    out = _embedding_lookup(x_flat, W_embed)
    return out.reshape(BATCH, SEQ, EMBED)


# speedup vs baseline: 1.5002x; 1.0013x over previous
import functools

import jax
import jax.numpy as jnp
from jax import lax
from jax.experimental import pallas as pl
from jax.experimental.pallas import tpu as pltpu
from jax.experimental.pallas import tpu_sc as plsc

BATCH = 4096
SEQ = 200
EMBED = 32
TOTAL = BATCH * SEQ

NUM_CORES = 2
NUM_SUBCORES = 16
NW = NUM_CORES * NUM_SUBCORES
PER_WORKER = TOTAL // NW
NB = 4
CHUNK = 640
NCHUNK = PER_WORKER // CHUNK


def _emb_body(idx_hbm, table_hbm, out_hbm, idx_v, *scr):
    rows = scr[:NB]
    gsem = scr[NB:2 * NB]
    ssem = scr[2 * NB:3 * NB]

    wid = lax.axis_index("s") * NUM_CORES + lax.axis_index("c")
    base = wid * PER_WORKER

    pltpu.sync_copy(idx_hbm.at[pl.ds(base, PER_WORKER)], idx_v)

    def gdesc(i, b):
        return pltpu.make_async_copy(
            table_hbm.at[idx_v.at[pl.ds(i * CHUNK, CHUNK)]], rows[b], gsem[b])

    def sdesc(i, b):
        return pltpu.make_async_copy(
            rows[b], out_hbm.at[pl.ds(base + i * CHUNK, CHUNK)], ssem[b])

    for k in range(NB - 1):
        gdesc(k, k).start()

    @pl.loop(0, NCHUNK, step=NB)
    def _(i):
        for b in range(NB):
            j = i + b
            gdesc(0, b).wait()
            sdesc(j, b).start()
            nxt = j + NB - 1
            pb = (b + NB - 1) % NB

            @pl.when(jnp.logical_and(nxt >= NB, nxt <= NCHUNK - 1))
            def _():
                sdesc(0, pb).wait()
                gdesc(nxt, pb).start()

            @pl.when(jnp.logical_and(nxt < NB, nxt <= NCHUNK - 1))
            def _():
                gdesc(nxt, pb).start()

    for b in range(NB):
        sdesc(0, b).wait()


@jax.jit
def _embedding_lookup(x_flat, table):
    mesh = plsc.VectorSubcoreMesh(core_axis_name="c", subcore_axis_name="s")
    kern = functools.partial(
        pl.kernel,
        mesh=mesh,
        out_type=jax.ShapeDtypeStruct((TOTAL, EMBED), jnp.float32),
        scratch_types=(
            [pltpu.VMEM((PER_WORKER,), jnp.int32)]
            + [pltpu.VMEM((CHUNK, EMBED), jnp.float32)] * NB
            + [pltpu.SemaphoreType.DMA] * (2 * NB)
        ),
        compiler_params=pltpu.CompilerParams(use_tc_tiling_on_sc=False),
    )(_emb_body)
    return kern(x_flat, table)


def kernel(x, W_embed):
    x_flat = x.reshape(TOTAL).astype(jnp.int32)
    out = _embedding_lookup(x_flat, W_embed)
    return out.reshape(BATCH, SEQ, EMBED)
